# baseline (speedup 1.0000x reference)
import functools

import jax
import jax.numpy as jnp
from jax import lax
from jax.experimental import pallas as pl
from jax.experimental.pallas import tpu as pltpu
from jax.experimental.pallas import tpu_sc as plsc

B, S, D = 2, 512, 768
H = 12
HD = D // H
E = 8
K = 2
HID = 1280
T = B * S
BM = 128
P = 3072
NT = P // BM
NW = 32


def _ln_body(x_ref, g_ref, b_ref, o_ref):
    xv = x_ref[...]
    mu = jnp.mean(xv, axis=1, keepdims=True)
    xc = xv - mu
    var = jnp.mean(xc * xc, axis=1, keepdims=True)
    o_ref[...] = (xc / jnp.sqrt(var + 1e-5) * g_ref[...] + b_ref[...]
                  ).astype(jnp.bfloat16)


def _ln1(xf, g, b):
    return pl.pallas_call(
        _ln_body,
        grid=(8,),
        in_specs=[
            pl.BlockSpec((T // 8, D), lambda i: (i, 0)),
            pl.BlockSpec((1, D), lambda i: (0, 0)),
            pl.BlockSpec((1, D), lambda i: (0, 0)),
        ],
        out_specs=pl.BlockSpec((T // 8, D), lambda i: (i, 0)),
        out_shape=jax.ShapeDtypeStruct((T, D), jnp.bfloat16),
    )(xf, g, b)


_SR = 4


def _attn_body(x_ref, wq_ref, wk_ref, wv_ref, pi_ref, c_ref, s_ref, o_ref):
    xn = x_ref[0]
    wcat = jnp.concatenate([wq_ref[...], wk_ref[...], wv_ref[...]], axis=0)
    qkv = lax.dot_general(xn, wcat, (((1,), (1,)), ((), ())),
                          preferred_element_type=jnp.float32)
    q0 = qkv[:, :HD]
    k0 = qkv[:, HD:2 * HD]
    v = qkv[:, 2 * HD:].astype(jnp.bfloat16)
    pi = pi_ref[...]
    q = lax.dot_general(q0.astype(jnp.bfloat16), pi,
                        (((1,), (0,)), ((), ())),
                        preferred_element_type=jnp.float32)
    k = lax.dot_general(k0.astype(jnp.bfloat16), pi,
                        (((1,), (0,)), ((), ())),
                        preferred_element_type=jnp.float32)
    c = c_ref[...]
    s = s_ref[...]
    hh = HD // 2
    qe, qo = q[:, :hh], q[:, hh:]
    ke, ko = k[:, :hh], k[:, hh:]
    qr = jnp.concatenate([qe * c - qo * s, qe * s + qo * c],
                         axis=1).astype(jnp.bfloat16)
    kr = jnp.concatenate([ke * c - ko * s, ke * s + ko * c],
                         axis=1).astype(jnp.bfloat16)
    sc = lax.dot_general(qr, kr, (((1,), (1,)), ((), ())),
                         preferred_element_type=jnp.float32) * (1.0 / 8.0)
    ri = lax.broadcasted_iota(jnp.int32, (S, S), 0)
    ci = lax.broadcasted_iota(jnp.int32, (S, S), 1)
    sc = jnp.where(ri >= ci, sc, jnp.finfo(jnp.float32).min)
    mx = jnp.max(sc, axis=1, keepdims=True)
    p = jnp.exp(sc - mx)
    inv = 1.0 / jnp.sum(p, axis=1, keepdims=True)
    ov = lax.dot_general(p.astype(jnp.bfloat16), v, (((1,), (0,)), ((), ())),
                         preferred_element_type=jnp.float32)
    o_ref[0, 0] = (ov * inv).astype(jnp.bfloat16)


def _attention(xn3, wqkv, pi64, cos, sin):
    wblk = lambda off: pl.BlockSpec(
        (HD, D), lambda bi, h, o=off: (o + h, 0))
    return pl.pallas_call(
        _attn_body,
        grid=(B, H),
        in_specs=[
            pl.BlockSpec((1, S, D), lambda bi, h: (bi, 0, 0)),
            wblk(0), wblk(H), wblk(2 * H),
            pl.BlockSpec((HD, HD), lambda bi, h: (0, 0)),
            pl.BlockSpec((S, HD // 2), lambda bi, h: (0, 0)),
            pl.BlockSpec((S, HD // 2), lambda bi, h: (0, 0)),
        ],
        out_specs=pl.BlockSpec((1, 1, S, HD), lambda bi, h: (h, bi, 0, 0)),
        out_shape=jax.ShapeDtypeStruct((H, B, S, HD), jnp.bfloat16),
    )(xn3, wqkv, wqkv, wqkv, pi64, cos, sin)


_CT = 128


def _post_body(x_ref, ao_ref, ow_ref, g_ref, b_ref, gw_ref,
               xmid_ref, xn_ref, lg_ref, ed_ref):
    acc = x_ref[...]
    for h in range(H):
        acc = acc + lax.dot_general(
            ao_ref[h], ow_ref[h], (((1,), (1,)), ((), ())),
            preferred_element_type=jnp.float32)
    xmid_ref[...] = acc
    mu = jnp.mean(acc, axis=1, keepdims=True)
    xc = acc - mu
    var = jnp.mean(xc * xc, axis=1, keepdims=True)
    xn = xc / jnp.sqrt(var + 1e-5) * g_ref[...] + b_ref[...]
    xnb = xn.astype(jnp.bfloat16)
    xn_ref[...] = xnb
    lg = lax.dot_general(xnb, gw_ref[...], (((1,), (1,)), ((), ())),
                         preferred_element_type=jnp.float32)
    lg_ref[...] = lg
    col = lax.broadcasted_iota(jnp.int32, (_CT, 128), 1)
    lgm = jnp.where(col < E, lg, -1e30)
    mx = jnp.max(lgm, axis=1, keepdims=True)
    p = jnp.exp(lgm - mx)
    p = p / jnp.sum(p, axis=1, keepdims=True)
    eds = jnp.sum(p, axis=0, keepdims=True)

    @pl.when(pl.program_id(0) == 0)
    def _():
        ed_ref[...] = jnp.zeros((1, 128), jnp.float32)

    ed_ref[...] += eds


def _post_attn(xf, ao3, ow3, g, b, gwp):
    nct = T // _CT
    return pl.pallas_call(
        _post_body,
        grid=(nct,),
        in_specs=[
            pl.BlockSpec((_CT, D), lambda i: (i, 0)),
            pl.BlockSpec((H, _CT, HD), lambda i: (0, i, 0)),
            pl.BlockSpec((H, D, HD), lambda i: (0, 0, 0)),
            pl.BlockSpec((1, D), lambda i: (0, 0)),
            pl.BlockSpec((1, D), lambda i: (0, 0)),
            pl.BlockSpec((128, D), lambda i: (0, 0)),
        ],
        out_specs=[
            pl.BlockSpec((_CT, D), lambda i: (i, 0)),
            pl.BlockSpec((_CT, D), lambda i: (i, 0)),
            pl.BlockSpec((_CT, 128), lambda i: (i, 0)),
            pl.BlockSpec((1, 128), lambda i: (0, 0)),
        ],
        out_shape=[
            jax.ShapeDtypeStruct((T, D), jnp.float32),
            jax.ShapeDtypeStruct((T, D), jnp.bfloat16),
            jax.ShapeDtypeStruct((T, 128), jnp.float32),
            jax.ShapeDtypeStruct((1, 128), jnp.float32),
        ],
    )(xf, ao3, ow3, g, b, gwp)


@functools.lru_cache(maxsize=None)
def _sc_gather_fn(n_idx, d):
    bpw = n_idx // NW
    mesh = plsc.VectorSubcoreMesh(core_axis_name="c", subcore_axis_name="s")

    @functools.partial(
        pl.kernel, mesh=mesh,
        out_type=jax.ShapeDtypeStruct((n_idx, d), jnp.float32),
        scratch_types=[
            pltpu.VMEM((bpw,), jnp.int32),
            pltpu.VMEM((bpw, d), jnp.float32),
            pltpu.SemaphoreType.DMA,
        ],
    )
    def gk(table_hbm, idx_hbm, out_hbm, idx_v, rows_v, sem):
        wid = lax.axis_index("s") * 2 + lax.axis_index("c")
        base = wid * bpw
        pltpu.sync_copy(idx_hbm.at[pl.ds(base, bpw)], idx_v)
        pltpu.async_copy(table_hbm.at[idx_v], rows_v, sem).wait()
        pltpu.sync_copy(rows_v, out_hbm.at[pl.ds(base, bpw)])

    return gk


def _gather_rows(table, idx):
    return _sc_gather_fn(idx.shape[0], table.shape[1])(table, idx)


def _ffn_body(te_ref, ta_ref, d2_ref, wf_ref, xn_ref, wg_ref, wu_ref, wd_ref,
              o_ref, wgb_ref, wub_ref, wdb_ref):
    i = pl.program_id(0)
    te = te_ref[i]
    prev = te_ref[jnp.maximum(i - 1, 0)]
    act = ta_ref[i]

    @pl.when(((i == 0) | (te != prev)) & (act != 0))
    def _():
        wgb_ref[...] = wg_ref[0].astype(jnp.bfloat16)
        wub_ref[...] = wu_ref[0].astype(jnp.bfloat16)
        wdb_ref[...] = wd_ref[0].astype(jnp.bfloat16)

    @pl.when(act != 0)
    def _():
        rid = i * BM + lax.broadcasted_iota(jnp.int32, (BM, 1), 0)
        d0 = d2_ref[0, :]
        d1 = d2_ref[1, :]
        oh0 = (d0[None, :] == rid).astype(jnp.float32)
        oh1 = (d1[None, :] == rid).astype(jnp.float32)
        ohs = oh0 + oh1
        xs = lax.dot_general(ohs.astype(jnp.bfloat16), xn_ref[...],
                             (((1,), (0,)), ((), ())),
                             preferred_element_type=jnp.float32
                             ).astype(jnp.bfloat16)
        wv = (lax.dot_general(oh0, wf_ref[0, :], (((1,), (0,)), ((), ())),
                              preferred_element_type=jnp.float32)
              + lax.dot_general(oh1, wf_ref[1, :], (((1,), (0,)), ((), ())),
                                preferred_element_type=jnp.float32))
        g = lax.dot_general(xs, wgb_ref[...], (((1,), (1,)), ((), ())),
                            preferred_element_type=jnp.float32)
        u = lax.dot_general(xs, wub_ref[...], (((1,), (1,)), ((), ())),
                            preferred_element_type=jnp.float32)
        h = (g * lax.logistic(g) * u).astype(jnp.bfloat16)
        o = lax.dot_general(h, wdb_ref[...], (((1,), (1,)), ((), ())),
                            preferred_element_type=jnp.float32)
        o_ref[...] = o * wv[:, None]

    @pl.when(act == 0)
    def _():
        o_ref[...] = jnp.zeros((BM, D), jnp.float32)


def _grouped_ffn(tile_e, tile_act, dest2, wf2, xn2b, wg, wu, wd):
    return pl.pallas_call(
        _ffn_body,
        grid_spec=pltpu.PrefetchScalarGridSpec(
            num_scalar_prefetch=2,
            grid=(NT,),
            in_specs=[
                pl.BlockSpec((K, T), lambda i, te, ta: (0, 0)),
                pl.BlockSpec((K, T), lambda i, te, ta: (0, 0)),
                pl.BlockSpec((T, D), lambda i, te, ta: (0, 0)),
                pl.BlockSpec((1, HID, D), lambda i, te, ta: (te[i], 0, 0)),
                pl.BlockSpec((1, HID, D), lambda i, te, ta: (te[i], 0, 0)),
                pl.BlockSpec((1, D, HID), lambda i, te, ta: (te[i], 0, 0)),
            ],
            out_specs=pl.BlockSpec((BM, D), lambda i, te, ta: (i, 0)),
            scratch_shapes=[
                pltpu.VMEM((HID, D), jnp.bfloat16),
                pltpu.VMEM((HID, D), jnp.bfloat16),
                pltpu.VMEM((D, HID), jnp.bfloat16),
            ],
        ),
        out_shape=jax.ShapeDtypeStruct((P, D), jnp.float32),
    )(tile_e, tile_act, dest2, wf2, xn2b, wg, wu, wd)


def _final_body(xm_ref, c0_ref, c1_ref, o_ref):
    o_ref[...] = xm_ref[...] + c0_ref[...] + c1_ref[...]


def _final_add(xmid, comb):
    return pl.pallas_call(
        _final_body,
        grid=(4,),
        in_specs=[
            pl.BlockSpec((T // 4, D), lambda i: (i, 0)),
            pl.BlockSpec((T // 4, D), lambda i: (i, 0)),
            pl.BlockSpec((T // 4, D), lambda i: (4 + i, 0)),
        ],
        out_specs=pl.BlockSpec((T // 4, D), lambda i: (i, 0)),
        out_shape=jax.ShapeDtypeStruct((T, D), jnp.float32),
    )(xmid, comb, comb)


def kernel(x, qkv_w, o_w, ln1_g, ln1_b, ln2_g, ln2_b, gate_w, wg, wu, wd):
    xf = x.reshape(T, D)

    inv_freq = 1.0 / (10000.0 ** (
        jnp.arange(0, HD, 2, dtype=jnp.float32) / HD))
    tt = jnp.arange(S, dtype=jnp.float32)
    freqs = tt[:, None] * inv_freq[None, :]
    cos = jnp.cos(freqs)
    sin = jnp.sin(freqs)

    wqkv = qkv_w.astype(jnp.bfloat16)
    ow3 = o_w.astype(jnp.bfloat16).reshape(D, H, HD).transpose(1, 0, 2)
    old = jnp.concatenate([jnp.arange(0, HD, 2, dtype=jnp.int32),
                           jnp.arange(1, HD, 2, dtype=jnp.int32)])
    pi64 = (old[:, None] ==
            jnp.arange(HD, dtype=jnp.int32)[None, :]).astype(jnp.bfloat16).T

    xnb = _ln1(xf, ln1_g.reshape(1, D), ln1_b.reshape(1, D))
    ao3 = _attention(xnb.reshape(B, S, D), wqkv, pi64, cos,
                     sin).reshape(H, T, HD)

    gwp = jnp.pad(gate_w, ((0, 128 - E), (0, 0))).astype(jnp.bfloat16)
    xmid, xn2b, logits, edp = _post_attn(
        xf, ao3, ow3, ln2_g.reshape(1, D), ln2_b.reshape(1, D), gwp)
    ed = edp[0, :E] * (1.0 / T)
    lb = jnp.float32(E) * jnp.sum(ed * ed)

    tw, ti = lax.top_k(logits[:, :E], K)
    tw = jax.nn.softmax(tw, axis=-1)
    e_flat = ti.reshape(-1).astype(jnp.int32)
    w_flat = tw.reshape(-1)
    onehot = (e_flat[:, None] == jnp.arange(E, dtype=jnp.int32)[None, :])
    onehot = onehot.astype(jnp.int32)
    rank = jnp.sum((jnp.cumsum(onehot, axis=0) - onehot) * onehot, axis=1)
    sizes = jnp.sum(onehot, axis=0)
    padded = ((sizes + BM - 1) // BM) * BM
    cum = jnp.cumsum(padded)
    offs = cum - padded
    dest = (offs[e_flat] + rank).astype(jnp.int32)
    dest2 = dest.reshape(T, K).T
    wf2 = w_flat.reshape(T, K).T
    tile_e = jnp.minimum(
        jnp.searchsorted(cum, jnp.arange(NT, dtype=jnp.int32) * BM,
                         side='right'),
        E - 1).astype(jnp.int32)
    ends = offs + sizes
    tile_act = (jnp.arange(NT, dtype=jnp.int32) * BM
                < ends[tile_e]).astype(jnp.int32)

    out_p = _grouped_ffn(tile_e, tile_act, dest2, wf2, xn2b, wg, wu, wd)
    comb = _gather_rows(out_p, dest2.reshape(K * T))
    y = _final_add(xmid, comb)

    return (y.reshape(B, S, D), lb, ed)


# speedup vs baseline: 1.1372x; 1.1372x over previous
import functools

import jax
import jax.numpy as jnp
from jax import lax
from jax.experimental import pallas as pl
from jax.experimental.pallas import tpu as pltpu
from jax.experimental.pallas import tpu_sc as plsc

B, S, D = 2, 512, 768
H = 12
HD = D // H
E = 8
K = 2
HID = 1280
T = B * S
BM = 256
P = 4096
NT = P // BM
NW = 32


def _ln_body(x_ref, g_ref, b_ref, o_ref):
    xv = x_ref[...]
    mu = jnp.mean(xv, axis=1, keepdims=True)
    xc = xv - mu
    var = jnp.mean(xc * xc, axis=1, keepdims=True)
    o_ref[...] = (xc / jnp.sqrt(var + 1e-5) * g_ref[...] + b_ref[...]
                  ).astype(jnp.bfloat16)


def _ln1(xf, g, b):
    return pl.pallas_call(
        _ln_body,
        grid=(8,),
        in_specs=[
            pl.BlockSpec((T // 8, D), lambda i: (i, 0)),
            pl.BlockSpec((1, D), lambda i: (0, 0)),
            pl.BlockSpec((1, D), lambda i: (0, 0)),
        ],
        out_specs=pl.BlockSpec((T // 8, D), lambda i: (i, 0)),
        out_shape=jax.ShapeDtypeStruct((T, D), jnp.bfloat16),
    )(xf, g, b)


_SR = 4


def _attn_body(x_ref, wq_ref, wk_ref, wv_ref, pi_ref, c_ref, s_ref, o_ref):
    xn = x_ref[0]
    wcat = jnp.concatenate([wq_ref[...], wk_ref[...], wv_ref[...]], axis=0)
    qkv = lax.dot_general(xn, wcat, (((1,), (1,)), ((), ())),
                          preferred_element_type=jnp.float32)
    q0 = qkv[:, :HD]
    k0 = qkv[:, HD:2 * HD]
    v = qkv[:, 2 * HD:].astype(jnp.bfloat16)
    pi = pi_ref[...]
    q = lax.dot_general(q0.astype(jnp.bfloat16), pi,
                        (((1,), (0,)), ((), ())),
                        preferred_element_type=jnp.float32)
    k = lax.dot_general(k0.astype(jnp.bfloat16), pi,
                        (((1,), (0,)), ((), ())),
                        preferred_element_type=jnp.float32)
    c = c_ref[...]
    s = s_ref[...]
    hh = HD // 2
    qe, qo = q[:, :hh], q[:, hh:]
    ke, ko = k[:, :hh], k[:, hh:]
    qr = jnp.concatenate([qe * c - qo * s, qe * s + qo * c],
                         axis=1).astype(jnp.bfloat16)
    kr = jnp.concatenate([ke * c - ko * s, ke * s + ko * c],
                         axis=1).astype(jnp.bfloat16)
    sc = lax.dot_general(qr, kr, (((1,), (1,)), ((), ())),
                         preferred_element_type=jnp.float32) * (1.0 / 8.0)
    ri = lax.broadcasted_iota(jnp.int32, (S, S), 0)
    ci = lax.broadcasted_iota(jnp.int32, (S, S), 1)
    sc = jnp.where(ri >= ci, sc, jnp.finfo(jnp.float32).min)
    mx = jnp.max(sc, axis=1, keepdims=True)
    p = jnp.exp(sc - mx)
    inv = 1.0 / jnp.sum(p, axis=1, keepdims=True)
    ov = lax.dot_general(p.astype(jnp.bfloat16), v, (((1,), (0,)), ((), ())),
                         preferred_element_type=jnp.float32)
    o_ref[0, 0] = (ov * inv).astype(jnp.bfloat16)


def _attention(xn3, wqkv, pi64, cos, sin):
    wblk = lambda off: pl.BlockSpec(
        (HD, D), lambda bi, h, o=off: (o + h, 0))
    return pl.pallas_call(
        _attn_body,
        grid=(B, H),
        in_specs=[
            pl.BlockSpec((1, S, D), lambda bi, h: (bi, 0, 0)),
            wblk(0), wblk(H), wblk(2 * H),
            pl.BlockSpec((HD, HD), lambda bi, h: (0, 0)),
            pl.BlockSpec((S, HD // 2), lambda bi, h: (0, 0)),
            pl.BlockSpec((S, HD // 2), lambda bi, h: (0, 0)),
        ],
        out_specs=pl.BlockSpec((1, 1, S, HD), lambda bi, h: (h, bi, 0, 0)),
        out_shape=jax.ShapeDtypeStruct((H, B, S, HD), jnp.bfloat16),
    )(xn3, wqkv, wqkv, wqkv, pi64, cos, sin)


_CT = 128


def _post_body(x_ref, ao_ref, ow_ref, g_ref, b_ref, gw_ref,
               xmid_ref, xn_ref, lg_ref, ed_ref):
    acc = x_ref[...]
    for h in range(H):
        acc = acc + lax.dot_general(
            ao_ref[h], ow_ref[h], (((1,), (1,)), ((), ())),
            preferred_element_type=jnp.float32)
    xmid_ref[...] = acc
    mu = jnp.mean(acc, axis=1, keepdims=True)
    xc = acc - mu
    var = jnp.mean(xc * xc, axis=1, keepdims=True)
    xn = xc / jnp.sqrt(var + 1e-5) * g_ref[...] + b_ref[...]
    xnb = xn.astype(jnp.bfloat16)
    xn_ref[...] = xnb
    lg = lax.dot_general(xnb, gw_ref[...], (((1,), (1,)), ((), ())),
                         preferred_element_type=jnp.float32)
    lg_ref[...] = lg
    col = lax.broadcasted_iota(jnp.int32, (_CT, 128), 1)
    lgm = jnp.where(col < E, lg, -1e30)
    mx = jnp.max(lgm, axis=1, keepdims=True)
    p = jnp.exp(lgm - mx)
    p = p / jnp.sum(p, axis=1, keepdims=True)
    eds = jnp.sum(p, axis=0, keepdims=True)

    @pl.when(pl.program_id(0) == 0)
    def _():
        ed_ref[...] = jnp.zeros((1, 128), jnp.float32)

    ed_ref[...] += eds


def _post_attn(xf, ao3, ow3, g, b, gwp):
    nct = T // _CT
    return pl.pallas_call(
        _post_body,
        grid=(nct,),
        in_specs=[
            pl.BlockSpec((_CT, D), lambda i: (i, 0)),
            pl.BlockSpec((H, _CT, HD), lambda i: (0, i, 0)),
            pl.BlockSpec((H, D, HD), lambda i: (0, 0, 0)),
            pl.BlockSpec((1, D), lambda i: (0, 0)),
            pl.BlockSpec((1, D), lambda i: (0, 0)),
            pl.BlockSpec((128, D), lambda i: (0, 0)),
        ],
        out_specs=[
            pl.BlockSpec((_CT, D), lambda i: (i, 0)),
            pl.BlockSpec((_CT, D), lambda i: (i, 0)),
            pl.BlockSpec((_CT, 128), lambda i: (i, 0)),
            pl.BlockSpec((1, 128), lambda i: (0, 0)),
        ],
        out_shape=[
            jax.ShapeDtypeStruct((T, D), jnp.float32),
            jax.ShapeDtypeStruct((T, D), jnp.bfloat16),
            jax.ShapeDtypeStruct((T, 128), jnp.float32),
            jax.ShapeDtypeStruct((1, 128), jnp.float32),
        ],
    )(xf, ao3, ow3, g, b, gwp)


@functools.lru_cache(maxsize=None)
def _sc_gather_fn(n_idx, d):
    bpw = n_idx // NW
    mesh = plsc.VectorSubcoreMesh(core_axis_name="c", subcore_axis_name="s")

    @functools.partial(
        pl.kernel, mesh=mesh,
        out_type=jax.ShapeDtypeStruct((n_idx, d), jnp.float32),
        scratch_types=[
            pltpu.VMEM((bpw,), jnp.int32),
            pltpu.VMEM((bpw, d), jnp.float32),
            pltpu.SemaphoreType.DMA,
        ],
    )
    def gk(table_hbm, idx_hbm, out_hbm, idx_v, rows_v, sem):
        wid = lax.axis_index("s") * 2 + lax.axis_index("c")
        base = wid * bpw
        pltpu.sync_copy(idx_hbm.at[pl.ds(base, bpw)], idx_v)
        pltpu.async_copy(table_hbm.at[idx_v], rows_v, sem).wait()
        pltpu.sync_copy(rows_v, out_hbm.at[pl.ds(base, bpw)])

    return gk


def _gather_rows(table, idx):
    return _sc_gather_fn(idx.shape[0], table.shape[1])(table, idx)


def _ffn_body(te_ref, ta_ref, d2_ref, wf_ref, xn_ref, wg_ref, wu_ref, wd_ref,
              o_ref, wgb_ref, wub_ref, wdb_ref):
    i = pl.program_id(0)
    te = te_ref[i]
    prev = te_ref[jnp.maximum(i - 1, 0)]
    act = ta_ref[i]

    @pl.when(((i == 0) | (te != prev)) & (act != 0))
    def _():
        wgb_ref[...] = wg_ref[0].astype(jnp.bfloat16)
        wub_ref[...] = wu_ref[0].astype(jnp.bfloat16)
        wdb_ref[...] = wd_ref[0].astype(jnp.bfloat16)

    @pl.when(act != 0)
    def _():
        rid = i * BM + lax.broadcasted_iota(jnp.int32, (BM, 1), 0)
        d0 = d2_ref[0, :]
        d1 = d2_ref[1, :]
        oh0 = (d0[None, :] == rid).astype(jnp.float32)
        oh1 = (d1[None, :] == rid).astype(jnp.float32)
        ohs = oh0 + oh1
        xs = lax.dot_general(ohs.astype(jnp.bfloat16), xn_ref[...],
                             (((1,), (0,)), ((), ())),
                             preferred_element_type=jnp.float32
                             ).astype(jnp.bfloat16)
        wv = (lax.dot_general(oh0, wf_ref[0, :], (((1,), (0,)), ((), ())),
                              preferred_element_type=jnp.float32)
              + lax.dot_general(oh1, wf_ref[1, :], (((1,), (0,)), ((), ())),
                                preferred_element_type=jnp.float32))
        g = lax.dot_general(xs, wgb_ref[...], (((1,), (1,)), ((), ())),
                            preferred_element_type=jnp.float32)
        u = lax.dot_general(xs, wub_ref[...], (((1,), (1,)), ((), ())),
                            preferred_element_type=jnp.float32)
        h = (g * lax.logistic(g) * u).astype(jnp.bfloat16)
        o = lax.dot_general(h, wdb_ref[...], (((1,), (1,)), ((), ())),
                            preferred_element_type=jnp.float32)
        o_ref[...] = o * wv[:, None]

    @pl.when(act == 0)
    def _():
        o_ref[...] = jnp.zeros((BM, D), jnp.float32)


def _grouped_ffn(tile_e, tile_act, dest2, wf2, xn2b, wg, wu, wd):
    return pl.pallas_call(
        _ffn_body,
        grid_spec=pltpu.PrefetchScalarGridSpec(
            num_scalar_prefetch=2,
            grid=(NT,),
            in_specs=[
                pl.BlockSpec((K, T), lambda i, te, ta: (0, 0)),
                pl.BlockSpec((K, T), lambda i, te, ta: (0, 0)),
                pl.BlockSpec((T, D), lambda i, te, ta: (0, 0)),
                pl.BlockSpec((1, HID, D), lambda i, te, ta: (te[i], 0, 0)),
                pl.BlockSpec((1, HID, D), lambda i, te, ta: (te[i], 0, 0)),
                pl.BlockSpec((1, D, HID), lambda i, te, ta: (te[i], 0, 0)),
            ],
            out_specs=pl.BlockSpec((BM, D), lambda i, te, ta: (i, 0)),
            scratch_shapes=[
                pltpu.VMEM((HID, D), jnp.bfloat16),
                pltpu.VMEM((HID, D), jnp.bfloat16),
                pltpu.VMEM((D, HID), jnp.bfloat16),
            ],
        ),
        out_shape=jax.ShapeDtypeStruct((P, D), jnp.float32),
    )(tile_e, tile_act, dest2, wf2, xn2b, wg, wu, wd)


def _final_body(xm_ref, c0_ref, c1_ref, o_ref):
    o_ref[...] = xm_ref[...] + c0_ref[...] + c1_ref[...]


def _final_add(xmid, comb):
    return pl.pallas_call(
        _final_body,
        grid=(4,),
        in_specs=[
            pl.BlockSpec((T // 4, D), lambda i: (i, 0)),
            pl.BlockSpec((T // 4, D), lambda i: (i, 0)),
            pl.BlockSpec((T // 4, D), lambda i: (4 + i, 0)),
        ],
        out_specs=pl.BlockSpec((T // 4, D), lambda i: (i, 0)),
        out_shape=jax.ShapeDtypeStruct((T, D), jnp.float32),
    )(xmid, comb, comb)


def kernel(x, qkv_w, o_w, ln1_g, ln1_b, ln2_g, ln2_b, gate_w, wg, wu, wd):
    xf = x.reshape(T, D)

    inv_freq = 1.0 / (10000.0 ** (
        jnp.arange(0, HD, 2, dtype=jnp.float32) / HD))
    tt = jnp.arange(S, dtype=jnp.float32)
    freqs = tt[:, None] * inv_freq[None, :]
    cos = jnp.cos(freqs)
    sin = jnp.sin(freqs)

    wqkv = qkv_w.astype(jnp.bfloat16)
    ow3 = o_w.astype(jnp.bfloat16).reshape(D, H, HD).transpose(1, 0, 2)
    old = jnp.concatenate([jnp.arange(0, HD, 2, dtype=jnp.int32),
                           jnp.arange(1, HD, 2, dtype=jnp.int32)])
    pi64 = (old[:, None] ==
            jnp.arange(HD, dtype=jnp.int32)[None, :]).astype(jnp.bfloat16).T

    xnb = _ln1(xf, ln1_g.reshape(1, D), ln1_b.reshape(1, D))
    ao3 = _attention(xnb.reshape(B, S, D), wqkv, pi64, cos,
                     sin).reshape(H, T, HD)

    gwp = jnp.pad(gate_w, ((0, 128 - E), (0, 0))).astype(jnp.bfloat16)
    xmid, xn2b, logits, edp = _post_attn(
        xf, ao3, ow3, ln2_g.reshape(1, D), ln2_b.reshape(1, D), gwp)
    ed = edp[0, :E] * (1.0 / T)
    lb = jnp.float32(E) * jnp.sum(ed * ed)

    tw, ti = lax.top_k(logits[:, :E], K)
    tw = jax.nn.softmax(tw, axis=-1)
    e_flat = ti.reshape(-1).astype(jnp.int32)
    w_flat = tw.reshape(-1)
    onehot = (e_flat[:, None] == jnp.arange(E, dtype=jnp.int32)[None, :])
    onehot = onehot.astype(jnp.int32)
    rank = jnp.sum((jnp.cumsum(onehot, axis=0) - onehot) * onehot, axis=1)
    sizes = jnp.sum(onehot, axis=0)
    padded = ((sizes + BM - 1) // BM) * BM
    cum = jnp.cumsum(padded)
    offs = cum - padded
    dest = (offs[e_flat] + rank).astype(jnp.int32)
    dest2 = dest.reshape(T, K).T
    wf2 = w_flat.reshape(T, K).T
    tile_e = jnp.minimum(
        jnp.searchsorted(cum, jnp.arange(NT, dtype=jnp.int32) * BM,
                         side='right'),
        E - 1).astype(jnp.int32)
    ends = offs + sizes
    tile_act = (jnp.arange(NT, dtype=jnp.int32) * BM
                < ends[tile_e]).astype(jnp.int32)

    out_p = _grouped_ffn(tile_e, tile_act, dest2, wf2, xn2b, wg, wu, wd)
    comb = _gather_rows(out_p, dest2.reshape(K * T))
    y = _final_add(xmid, comb)

    return (y.reshape(B, S, D), lb, ed)
